# baseline (speedup 1.0000x reference)
import functools

import jax
import jax.numpy as jnp
from jax import lax
from jax.experimental import pallas as pl
from jax.experimental.pallas import tpu as pltpu
from jax.experimental.pallas import tpu_sc as plsc

NC = 2
NS = 16
NW = NC * NS
L = 16


def _sc_mesh():
    return plsc.VectorSubcoreMesh(core_axis_name="c", subcore_axis_name="s")


def _sc_deg(colp, ewp, n_pad):
    e_pad = colp.shape[0]
    stripe = e_pad // NW
    slc = n_pad // NS

    @functools.partial(
        pl.kernel,
        out_type=(
            jax.ShapeDtypeStruct((n_pad,), jnp.float32),
            jax.ShapeDtypeStruct((n_pad,), jnp.float32),
        ),
        mesh=_sc_mesh(),
        compiler_params=pltpu.CompilerParams(needs_layout_passes=False),
        scratch_types=[
            pltpu.VMEM((stripe,), jnp.int32),
            pltpu.VMEM((stripe,), jnp.float32),
            pltpu.VMEM((n_pad,), jnp.float32),
            pltpu.VMEM((NS, slc), jnp.float32),
            pltpu.VMEM((slc,), jnp.float32),
            pltpu.VMEM_SHARED((NS, n_pad), jnp.float32),
        ],
    )
    def k(col_hbm, ew_hbm, deg0_hbm, deg1_hbm, col_v, ew_v, hist, buf, res, shared):
        c = lax.axis_index("c")
        s = lax.axis_index("s")
        w = c * NS + s
        pltpu.sync_copy(col_hbm.at[pl.ds(w * stripe, stripe)], col_v)
        pltpu.sync_copy(ew_hbm.at[pl.ds(w * stripe, stripe)], ew_v)

        z = jnp.zeros((L,), jnp.float32)

        def zero_b(i, _):
            hist[pl.ds(i * L, L)] = z
            return 0

        lax.fori_loop(0, n_pad // L, zero_b, 0)

        def acc_b(i, _):
            cols = col_v[pl.ds(i * L, L)]
            ews = ew_v[pl.ds(i * L, L)]
            plsc.addupdate_scatter(hist, [cols], ews)
            return 0

        lax.fori_loop(0, stripe // L, acc_b, 0)

        pltpu.sync_copy(hist, shared.at[s])
        plsc.subcore_barrier()
        for t in range(NS):
            pltpu.sync_copy(shared.at[t, pl.ds(s * slc, slc)], buf.at[t])

        def sum_b(kk, _):
            v = buf[0, pl.ds(kk * L, L)]
            for t in range(1, NS):
                v = v + buf[t, pl.ds(kk * L, L)]
            res[pl.ds(kk * L, L)] = v
            return 0

        lax.fori_loop(0, slc // L, sum_b, 0)

        @pl.when(c == 0)
        def _():
            pltpu.sync_copy(res, deg0_hbm.at[pl.ds(s * slc, slc)])

        @pl.when(c == 1)
        def _():
            pltpu.sync_copy(res, deg1_hbm.at[pl.ds(s * slc, slc)])

    return k(colp, ewp)


def _tc_matmul_scale(xp, W, deg0c, deg1c):
    n_pad, d_in = xp.shape
    d_out = W.shape[1]
    R = 128
    grid = n_pad // R

    def body(x_ref, w_ref, d0_ref, d1_ref, g_ref):
        dis = lax.rsqrt(d0_ref[...] + d1_ref[...] + 1.0)
        h = jnp.dot(x_ref[...], w_ref[...], preferred_element_type=jnp.float32)
        g_ref[...] = h * dis

    return pl.pallas_call(
        body,
        grid=(grid,),
        in_specs=[
            pl.BlockSpec((R, d_in), lambda i: (i, 0)),
            pl.BlockSpec((d_in, d_out), lambda i: (0, 0)),
            pl.BlockSpec((R, 1), lambda i: (i, 0)),
            pl.BlockSpec((R, 1), lambda i: (i, 0)),
        ],
        out_specs=pl.BlockSpec((R, d_out), lambda i: (i, 0)),
        out_shape=jax.ShapeDtypeStruct((n_pad, d_out), jnp.float32),
    )(xp, W, deg0c, deg1c)


def _sc_scatter(rowp, colp, ewp, g, n_pad, magic, shift):
    e_pad = rowp.shape[0]
    d = g.shape[1]
    rows_own = n_pad // NW
    B = 64
    CH = 4096
    assert e_pad % CH == 0
    spad = CH + 5 * B + 2 * L

    @functools.partial(
        pl.kernel,
        out_type=jax.ShapeDtypeStruct((n_pad * d,), jnp.float32),
        mesh=_sc_mesh(),
        compiler_params=pltpu.CompilerParams(needs_layout_passes=False),
        scratch_types=[
            pltpu.VMEM((spad,), jnp.int32),
            pltpu.VMEM((spad,), jnp.int32),
            pltpu.VMEM((spad,), jnp.float32),
            pltpu.VMEM((B,), jnp.int32),
            pltpu.VMEM((B,), jnp.int32),
            pltpu.VMEM((B, d), jnp.float32),
            pltpu.VMEM((B, d), jnp.float32),
            pltpu.VMEM((rows_own * d,), jnp.float32),
            pltpu.SemaphoreType.DMA,
            pltpu.SemaphoreType.DMA,
        ],
    )
    def k(row_hbm, col_hbm, ew_hbm, g_hbm, s_hbm,
          row_v, col_v, ew_v, bri0, bri1, rows0, rows1, acc, sem0, sem1):
        c = lax.axis_index("c")
        s = lax.axis_index("s")
        w = c * NS + s

        z = jnp.zeros((L,), jnp.float32)
        zi = jnp.zeros((L,), jnp.int32)
        lanes = lax.iota(jnp.int32, L)

        def zero_b(i, _):
            acc[pl.ds(i * L, L)] = z
            return 0

        lax.fori_loop(0, rows_own * d // L, zero_b, 0)

        def chunk_body(ch, _):
            eo = ch * CH
            pltpu.sync_copy(row_hbm.at[pl.ds(eo, CH)], row_v.at[pl.ds(0, CH)])
            pltpu.sync_copy(col_hbm.at[pl.ds(eo, CH)], col_v.at[pl.ds(0, CH)])
            pltpu.sync_copy(ew_hbm.at[pl.ds(eo, CH)], ew_v.at[pl.ds(0, CH)])

            def compact(i, ptr):
                cols = col_v[pl.ds(i * L, L)]
                rws = row_v[pl.ds(i * L, L)]
                ews = ew_v[pl.ds(i * L, L)]
                own = jnp.right_shift(cols * magic, shift)
                loc = cols - own * rows_own
                m = own == w
                mi = m.astype(jnp.int32)
                idx = jnp.full((L,), ptr, jnp.int32) + plsc.cumsum(mi) - 1
                plsc.store_scatter(row_v, [idx], rws, mask=m)
                plsc.store_scatter(col_v, [idx], loc, mask=m)
                plsc.store_scatter(ew_v, [idx], ews, mask=m)
                return ptr + jnp.sum(mi)

            cnt = lax.fori_loop(0, CH // L, compact, jnp.int32(0))

            tail0 = (cnt // L) * L
            mt = lanes >= (cnt - tail0)
            plsc.store_scatter(row_v, [tail0 + lanes], zi, mask=mt)
            plsc.store_scatter(col_v, [tail0 + lanes], zi, mask=mt)
            plsc.store_scatter(ew_v, [tail0 + lanes], z, mask=mt)
            for t in range(1, 4 * (B // L) + 1):
                row_v[pl.ds(tail0 + t * L, L)] = zi
                col_v[pl.ds(tail0 + t * L, L)] = zi
                ew_v[pl.ds(tail0 + t * L, L)] = z

            nb = (cnt + B - 1) // B
            nbp = jnp.maximum((nb + 1) // 2, 1)

            def issue(bi, bri_ref, rows_ref, sem_ref):
                o = bi * B
                for kk in range(B // L):
                    bri_ref[pl.ds(kk * L, L)] = row_v[pl.ds(o + kk * L, L)]
                pltpu.async_copy(g_hbm.at[bri_ref], rows_ref, sem_ref)

            def wait(bri_ref, rows_ref, sem_ref):
                pltpu.make_async_copy(g_hbm.at[bri_ref], rows_ref, sem_ref).wait()

            def process(bi, rows_ref):
                o = bi * B

                def edge2(j2, _):
                    j0 = j2 * 2
                    j1 = j0 + 1
                    f0 = jnp.full((L,), o + j0, jnp.int32)
                    f1 = jnp.full((L,), o + j1, jnp.int32)
                    ew0 = plsc.load_gather(ew_v, [f0])
                    ew1 = plsc.load_gather(ew_v, [f1])
                    base0 = plsc.load_gather(col_v, [f0]) * d + lanes
                    base1 = plsc.load_gather(col_v, [f1]) * d + lanes
                    for kk in range(d // L):
                        v0 = rows_ref[j0, pl.ds(kk * L, L)] * ew0
                        v1 = rows_ref[j1, pl.ds(kk * L, L)] * ew1
                        plsc.addupdate_scatter(acc, [base0 + kk * L], v0)
                        plsc.addupdate_scatter(acc, [base1 + kk * L], v1)
                    return 0

                lax.fori_loop(0, B // 2, edge2, 0)

            issue(0, bri0, rows0, sem0)

            def pair(p, _):
                b0 = 2 * p
                issue(b0 + 1, bri1, rows1, sem1)
                wait(bri0, rows0, sem0)
                process(b0, rows0)
                issue(b0 + 2, bri0, rows0, sem0)
                wait(bri1, rows1, sem1)
                process(b0 + 1, rows1)
                return 0

            lax.fori_loop(0, nbp, pair, 0)
            wait(bri0, rows0, sem0)
            return 0

        lax.fori_loop(0, e_pad // CH, chunk_body, 0)

        pltpu.sync_copy(acc, s_hbm.at[pl.ds(w * rows_own * d, rows_own * d)])

    return k(rowp, colp, ewp, g)


def _tc_epilogue(S, g, deg0c, deg1c, b2):
    n_pad, d = S.shape
    R = 128
    grid = n_pad // R

    def body(s_ref, g_ref, d0_ref, d1_ref, b_ref, o_ref):
        dis = lax.rsqrt(d0_ref[...] + d1_ref[...] + 1.0)
        o_ref[...] = dis * (s_ref[...] + g_ref[...]) + b_ref[...]

    return pl.pallas_call(
        body,
        grid=(grid,),
        in_specs=[
            pl.BlockSpec((R, d), lambda i: (i, 0)),
            pl.BlockSpec((R, d), lambda i: (i, 0)),
            pl.BlockSpec((R, 1), lambda i: (i, 0)),
            pl.BlockSpec((R, 1), lambda i: (i, 0)),
            pl.BlockSpec((1, d), lambda i: (0, 0)),
        ],
        out_specs=pl.BlockSpec((R, d), lambda i: (i, 0)),
        out_shape=jax.ShapeDtypeStruct((n_pad, d), jnp.float32),
    )(S, g, deg0c, deg1c, b2)


def _magic_div(d, nmax):
    shift = 1
    while (1 << shift) < d * nmax:
        shift += 1
    for sh in range(d.bit_length(), shift + 1):
        m = -(-(1 << sh) // d)
        e = m * d - (1 << sh)
        if e == 0 or nmax < (1 << sh) // e:
            if m * nmax < (1 << 31):
                return m, sh
    raise AssertionError("no magic divider")


def kernel(x, edge_index, edge_weight, W, b):
    n, d_in = x.shape
    d_out = W.shape[1]
    e = edge_index.shape[1]

    n_pad = ((n + NW * L - 1) // (NW * L)) * (NW * L)
    if n_pad % 128:
        n_pad = ((n_pad + 127) // 128) * 128
    e_pad = ((e + 4095) // 4096) * 4096
    magic, shift = _magic_div(n_pad // NW, n_pad)

    row = edge_index[0]
    col = edge_index[1]
    pe = e_pad - e
    rowp = jnp.concatenate([row, jnp.zeros((pe,), row.dtype)])
    colp = jnp.concatenate([col, jnp.full((pe,), n, col.dtype)])
    ewp = jnp.concatenate([edge_weight, jnp.zeros((pe,), edge_weight.dtype)])
    xp = jnp.concatenate([x, jnp.zeros((n_pad - n, d_in), x.dtype)])

    deg0, deg1 = _sc_deg(colp, ewp, n_pad)
    deg0c = deg0.reshape(n_pad, 1)
    deg1c = deg1.reshape(n_pad, 1)

    g = _tc_matmul_scale(xp, W, deg0c, deg1c)
    S = _sc_scatter(rowp, colp, ewp, g, n_pad, magic, shift).reshape(n_pad, d_out)
    out = _tc_epilogue(S, g, deg0c, deg1c, b.reshape(1, d_out))
    return out[:n]


# speedup vs baseline: 7.2826x; 7.2826x over previous
import functools

import jax
import jax.numpy as jnp
from jax import lax
from jax.experimental import pallas as pl
from jax.experimental.pallas import tpu as pltpu
from jax.experimental.pallas import tpu_sc as plsc

NC = 2
NS = 16
NW = NC * NS
L = 16


def _sc_mesh():
    return plsc.VectorSubcoreMesh(core_axis_name="c", subcore_axis_name="s")


def _sc_deg(colp, ewp, n_pad):
    e_pad = colp.shape[0]
    stripe = e_pad // NW
    slc = n_pad // NS

    @functools.partial(
        pl.kernel,
        out_type=(
            jax.ShapeDtypeStruct((n_pad,), jnp.float32),
            jax.ShapeDtypeStruct((n_pad,), jnp.float32),
        ),
        mesh=_sc_mesh(),
        compiler_params=pltpu.CompilerParams(needs_layout_passes=False),
        scratch_types=[
            pltpu.VMEM((stripe,), jnp.int32),
            pltpu.VMEM((stripe,), jnp.float32),
            pltpu.VMEM((n_pad,), jnp.float32),
            pltpu.VMEM((NS, slc), jnp.float32),
            pltpu.VMEM((slc,), jnp.float32),
            pltpu.VMEM_SHARED((NS, n_pad), jnp.float32),
        ],
    )
    def k(col_hbm, ew_hbm, deg0_hbm, deg1_hbm, col_v, ew_v, hist, buf, res, shared):
        c = lax.axis_index("c")
        s = lax.axis_index("s")
        w = c * NS + s
        pltpu.sync_copy(col_hbm.at[pl.ds(w * stripe, stripe)], col_v)
        pltpu.sync_copy(ew_hbm.at[pl.ds(w * stripe, stripe)], ew_v)

        z = jnp.zeros((L,), jnp.float32)

        def zero_b(i, _):
            hist[pl.ds(i * L, L)] = z
            return 0

        lax.fori_loop(0, n_pad // L, zero_b, 0)

        def acc_b(i, _):
            cols = col_v[pl.ds(i * L, L)]
            ews = ew_v[pl.ds(i * L, L)]
            plsc.addupdate_scatter(hist, [cols], ews)
            return 0

        lax.fori_loop(0, stripe // L, acc_b, 0)

        pltpu.sync_copy(hist, shared.at[s])
        plsc.subcore_barrier()
        for t in range(NS):
            pltpu.sync_copy(shared.at[t, pl.ds(s * slc, slc)], buf.at[t])

        def sum_b(kk, _):
            v = buf[0, pl.ds(kk * L, L)]
            for t in range(1, NS):
                v = v + buf[t, pl.ds(kk * L, L)]
            res[pl.ds(kk * L, L)] = v
            return 0

        lax.fori_loop(0, slc // L, sum_b, 0)

        @pl.when(c == 0)
        def _():
            pltpu.sync_copy(res, deg0_hbm.at[pl.ds(s * slc, slc)])

        @pl.when(c == 1)
        def _():
            pltpu.sync_copy(res, deg1_hbm.at[pl.ds(s * slc, slc)])

    return k(colp, ewp)


def _tc_matmul_scale(xp, W, deg0c, deg1c):
    n_pad, d_in = xp.shape
    d_out = W.shape[1]
    R = 128
    grid = n_pad // R

    def body(x_ref, w_ref, d0_ref, d1_ref, g_ref):
        dis = lax.rsqrt(d0_ref[...] + d1_ref[...] + 1.0)
        h = jnp.dot(x_ref[...], w_ref[...], preferred_element_type=jnp.float32)
        g_ref[...] = h * dis

    return pl.pallas_call(
        body,
        grid=(grid,),
        in_specs=[
            pl.BlockSpec((R, d_in), lambda i: (i, 0)),
            pl.BlockSpec((d_in, d_out), lambda i: (0, 0)),
            pl.BlockSpec((R, 1), lambda i: (i, 0)),
            pl.BlockSpec((R, 1), lambda i: (i, 0)),
        ],
        out_specs=pl.BlockSpec((R, d_out), lambda i: (i, 0)),
        out_shape=jax.ShapeDtypeStruct((n_pad, d_out), jnp.float32),
    )(xp, W, deg0c, deg1c)


def _sc_scatter(rowp, colp, ewp, g, n_pad, magic, shift):
    e_pad = rowp.shape[0]
    d = g.shape[1]
    rows_own = n_pad // NW
    B = 64
    CH = 4096
    assert e_pad % CH == 0
    spad = CH + 5 * B + 2 * L

    @functools.partial(
        pl.kernel,
        out_type=jax.ShapeDtypeStruct((n_pad * d,), jnp.float32),
        mesh=_sc_mesh(),
        compiler_params=pltpu.CompilerParams(needs_layout_passes=False),
        scratch_types=[
            pltpu.VMEM((spad,), jnp.int32),
            pltpu.VMEM((spad,), jnp.int32),
            pltpu.VMEM((spad,), jnp.float32),
            pltpu.VMEM((B,), jnp.int32),
            pltpu.VMEM((B,), jnp.int32),
            pltpu.VMEM((B, d), jnp.float32),
            pltpu.VMEM((B, d), jnp.float32),
            pltpu.VMEM((rows_own * d,), jnp.float32),
            pltpu.SemaphoreType.DMA,
            pltpu.SemaphoreType.DMA,
        ],
    )
    def k(row_hbm, col_hbm, ew_hbm, g_hbm, s_hbm,
          row_v, col_v, ew_v, bri0, bri1, rows0, rows1, acc, sem0, sem1):
        c = lax.axis_index("c")
        s = lax.axis_index("s")
        w = c * NS + s

        z = jnp.zeros((L,), jnp.float32)
        zi = jnp.zeros((L,), jnp.int32)
        lanes = lax.iota(jnp.int32, L)

        def zero_b(i, _):
            acc[pl.ds(i * L, L)] = z
            return 0

        lax.fori_loop(0, rows_own * d // L, zero_b, 0)

        def chunk_body(ch, _):
            eo = ch * CH
            pltpu.sync_copy(row_hbm.at[pl.ds(eo, CH)], row_v.at[pl.ds(0, CH)])
            pltpu.sync_copy(col_hbm.at[pl.ds(eo, CH)], col_v.at[pl.ds(0, CH)])
            pltpu.sync_copy(ew_hbm.at[pl.ds(eo, CH)], ew_v.at[pl.ds(0, CH)])

            def compact(i, ptr):
                cols = col_v[pl.ds(i * L, L)]
                rws = row_v[pl.ds(i * L, L)]
                ews = ew_v[pl.ds(i * L, L)]
                own = jnp.right_shift(cols * magic, shift)
                loc = cols - own * rows_own
                m = own == w
                mi = m.astype(jnp.int32)
                idx = jnp.full((L,), ptr, jnp.int32) + plsc.cumsum(mi) - 1
                plsc.store_scatter(row_v, [idx], rws, mask=m)
                plsc.store_scatter(col_v, [idx], loc, mask=m)
                plsc.store_scatter(ew_v, [idx], ews, mask=m)
                return ptr + jnp.sum(mi)

            cnt = lax.fori_loop(0, CH // L, compact, jnp.int32(0))

            tail0 = (cnt // L) * L
            mt = lanes >= (cnt - tail0)
            plsc.store_scatter(row_v, [tail0 + lanes], lanes * L, mask=mt)
            plsc.store_scatter(col_v, [tail0 + lanes], zi, mask=mt)
            plsc.store_scatter(ew_v, [tail0 + lanes], z, mask=mt)
            for t in range(1, 4 * (B // L) + 1):
                row_v[pl.ds(tail0 + t * L, L)] = lanes * L + t
                col_v[pl.ds(tail0 + t * L, L)] = zi
                ew_v[pl.ds(tail0 + t * L, L)] = z

            nb = (cnt + B - 1) // B
            nbp = jnp.maximum((nb + 1) // 2, 1)

            def issue(bi, bri_ref, rows_ref, sem_ref):
                o = bi * B
                for kk in range(B // L):
                    bri_ref[pl.ds(kk * L, L)] = row_v[pl.ds(o + kk * L, L)]
                pltpu.async_copy(g_hbm.at[bri_ref], rows_ref, sem_ref)

            def wait(bri_ref, rows_ref, sem_ref):
                pltpu.make_async_copy(g_hbm.at[bri_ref], rows_ref, sem_ref).wait()

            def process(bi, rows_ref):
                o = bi * B

                def edge2(j2, _):
                    j0 = j2 * 2
                    j1 = j0 + 1
                    f0 = jnp.full((L,), o + j0, jnp.int32)
                    f1 = jnp.full((L,), o + j1, jnp.int32)
                    ew0 = plsc.load_gather(ew_v, [f0])
                    ew1 = plsc.load_gather(ew_v, [f1])
                    base0 = plsc.load_gather(col_v, [f0]) * d + lanes
                    base1 = plsc.load_gather(col_v, [f1]) * d + lanes
                    for kk in range(d // L):
                        v0 = rows_ref[j0, pl.ds(kk * L, L)] * ew0
                        v1 = rows_ref[j1, pl.ds(kk * L, L)] * ew1
                        plsc.addupdate_scatter(acc, [base0 + kk * L], v0)
                        plsc.addupdate_scatter(acc, [base1 + kk * L], v1)
                    return 0

                lax.fori_loop(0, B // 2, edge2, 0)

            issue(0, bri0, rows0, sem0)

            def pair(p, _):
                b0 = 2 * p
                issue(b0 + 1, bri1, rows1, sem1)
                wait(bri0, rows0, sem0)
                process(b0, rows0)
                issue(b0 + 2, bri0, rows0, sem0)
                wait(bri1, rows1, sem1)
                process(b0 + 1, rows1)
                return 0

            lax.fori_loop(0, nbp, pair, 0)
            wait(bri0, rows0, sem0)
            return 0

        lax.fori_loop(0, e_pad // CH, chunk_body, 0)

        pltpu.sync_copy(acc, s_hbm.at[pl.ds(w * rows_own * d, rows_own * d)])

    return k(rowp, colp, ewp, g)


def _tc_epilogue(S, g, deg0c, deg1c, b2):
    n_pad, d = S.shape
    R = 128
    grid = n_pad // R

    def body(s_ref, g_ref, d0_ref, d1_ref, b_ref, o_ref):
        dis = lax.rsqrt(d0_ref[...] + d1_ref[...] + 1.0)
        o_ref[...] = dis * (s_ref[...] + g_ref[...]) + b_ref[...]

    return pl.pallas_call(
        body,
        grid=(grid,),
        in_specs=[
            pl.BlockSpec((R, d), lambda i: (i, 0)),
            pl.BlockSpec((R, d), lambda i: (i, 0)),
            pl.BlockSpec((R, 1), lambda i: (i, 0)),
            pl.BlockSpec((R, 1), lambda i: (i, 0)),
            pl.BlockSpec((1, d), lambda i: (0, 0)),
        ],
        out_specs=pl.BlockSpec((R, d), lambda i: (i, 0)),
        out_shape=jax.ShapeDtypeStruct((n_pad, d), jnp.float32),
    )(S, g, deg0c, deg1c, b2)


def _magic_div(d, nmax):
    shift = 1
    while (1 << shift) < d * nmax:
        shift += 1
    for sh in range(d.bit_length(), shift + 1):
        m = -(-(1 << sh) // d)
        e = m * d - (1 << sh)
        if e == 0 or nmax < (1 << sh) // e:
            if m * nmax < (1 << 31):
                return m, sh
    raise AssertionError("no magic divider")


def kernel(x, edge_index, edge_weight, W, b):
    n, d_in = x.shape
    d_out = W.shape[1]
    e = edge_index.shape[1]

    n_pad = ((n + NW * L - 1) // (NW * L)) * (NW * L)
    if n_pad % 128:
        n_pad = ((n_pad + 127) // 128) * 128
    e_pad = ((e + 4095) // 4096) * 4096
    magic, shift = _magic_div(n_pad // NW, n_pad)

    row = edge_index[0]
    col = edge_index[1]
    pe = e_pad - e
    rowp = jnp.concatenate([row, jnp.zeros((pe,), row.dtype)])
    colp = jnp.concatenate([col, jnp.full((pe,), n, col.dtype)])
    ewp = jnp.concatenate([edge_weight, jnp.zeros((pe,), edge_weight.dtype)])
    xp = jnp.concatenate([x, jnp.zeros((n_pad - n, d_in), x.dtype)])

    deg0, deg1 = _sc_deg(colp, ewp, n_pad)
    deg0c = deg0.reshape(n_pad, 1)
    deg1c = deg1.reshape(n_pad, 1)

    g = _tc_matmul_scale(xp, W, deg0c, deg1c)
    S = _sc_scatter(rowp, colp, ewp, g, n_pad, magic, shift).reshape(n_pad, d_out)
    out = _tc_epilogue(S, g, deg0c, deg1c, b.reshape(1, d_out))
    return out[:n]
